# baseline (speedup 1.0000x reference)
import jax
import jax.numpy as jnp
import numpy as np
from jax import lax
from jax.experimental import pallas as pl
from jax.experimental.pallas import tpu as pltpu
from jax.experimental.pallas import tpu_sc as plsc

H = W = 32
C = 768
NBOX = 6
B = 8
_EPS32 = float(np.finfo(np.float32).eps)

_NW = 32
_PW = (H * W) // _NW
_CHUNK = _PW * C
_XWORDS = B * H * W * C
_GRP = 4


def _wmat(lo, hi):
    n = (hi - lo + 1).astype(jnp.float32)
    inv = 32.0 / n
    r = jax.lax.broadcasted_iota(jnp.int32, (32, 32), 1)
    i_in = jax.lax.broadcasted_iota(jnp.int32, (32, 32), 0).astype(jnp.float32)
    j = jnp.clip(r - lo, 0, 31).astype(jnp.float32)
    sample_f = (j + 0.5) * inv - 0.5
    wt = jnp.maximum(0.0, 1.0 - jnp.abs(sample_f - i_in) / inv)
    tot = jnp.sum(wt, axis=0, keepdims=True)
    wt = jnp.where(jnp.abs(tot) > 1000.0 * _EPS32,
                   wt / jnp.where(tot != 0.0, tot, 1.0), 0.0)
    wt = jnp.where((sample_f >= -0.5) & (sample_f <= 31.5), wt, 0.0)
    return wt


def _overlay_body(y_ref, base_ref, e_ref, comb_ref):
    rr = jax.lax.broadcasted_iota(jnp.int32, (H, W), 0)
    cc = jax.lax.broadcasted_iota(jnp.int32, (H, W), 1)
    applied = jnp.zeros((H, W), jnp.float32)
    comb = jnp.zeros((H, W, C), jnp.float32)
    gxs, gys, masks = [], [], []
    for i in range(NBOX):
        b0 = y_ref[i, 0]
        b1 = y_ref[i, 1]
        b2 = y_ref[i, 2]
        b3 = y_ref[i, 3]
        valid = ((b0 >= 0) & (b1 >= 0) & (b2 >= 0) & (b3 >= 0)).astype(jnp.float32)
        x1g = jnp.clip(jnp.floor(b0.astype(jnp.float32) * (1.0 / 16.0)), 0.0, 31.0)
        y1g = jnp.clip(jnp.floor(b1.astype(jnp.float32) * (1.0 / 16.0)), 0.0, 31.0)
        x2g = jnp.clip(jnp.floor(b2.astype(jnp.float32) * (1.0 / 16.0)), 0.0, 31.0)
        y2g = jnp.clip(jnp.floor(b3.astype(jnp.float32) * (1.0 / 16.0)), 0.0, 31.0)
        x_min = jnp.minimum(x1g, x2g).astype(jnp.int32)
        x_max = jnp.maximum(x1g, x2g).astype(jnp.int32)
        y_min = jnp.minimum(y1g, y2g).astype(jnp.int32)
        y_max = jnp.maximum(y1g, y2g).astype(jnp.int32)

        box_mask = ((rr >= y_min) & (rr <= y_max) &
                    (cc >= x_min) & (cc <= x_max)).astype(jnp.float32) * valid
        new_mask = box_mask * (1.0 - applied)
        applied = applied + new_mask
        masks.append(new_mask)
        gxs.append(_wmat(x_min, x_max))
        gys.append(_wmat(y_min, y_max))

    gxcat = jnp.concatenate(gxs, axis=1).astype(jnp.bfloat16)
    v = jax.lax.dot_general(gxcat, base_ref[...], (((0,), (0,)), ((), ())),
                            preferred_element_type=jnp.float32)
    v = v.astype(jnp.bfloat16)
    v4t = jnp.swapaxes(v.reshape(NBOX, W, 32, C), 1, 2)
    for i in range(NBOX):
        w = jax.lax.dot_general(gys[i].astype(jnp.bfloat16),
                                v4t[i].reshape(32, W * C),
                                (((0,), (0,)), ((), ())),
                                preferred_element_type=jnp.float32)
        m2 = jax.lax.dot_general(masks[i].astype(jnp.bfloat16), e_ref[...],
                                 (((1,), (0,)), ((), ())),
                                 preferred_element_type=jnp.float32)
        comb = comb + (w * m2).reshape(H, W, C)
    comb_ref[...] = comb.reshape(H * W, C)


def _overlay(y32, base_jic, expand):
    return pl.pallas_call(
        _overlay_body,
        grid=(1,),
        in_specs=[
            pl.BlockSpec(memory_space=pltpu.SMEM),
            pl.BlockSpec((32, 32 * C), lambda b: (0, 0)),
            pl.BlockSpec((32, 32 * C), lambda b: (0, 0)),
        ],
        out_specs=pl.BlockSpec((H * W, C), lambda b: (0, 0)),
        out_shape=jax.ShapeDtypeStruct((H * W, C), jnp.float32),
    )(y32, base_jic, expand)


def _sc_add_body(x_hbm, comb_hbm, out_hbm, comb_v, b0,
                 csem, is0, os0):
    bufs = (b0,)
    isems = (is0,)
    osems = (os0,)
    cid = lax.axis_index("c")
    sid = lax.axis_index("s")
    wid = sid * 2 + cid
    rbase = wid * _PW

    pltpu.async_copy(comb_hbm.at[pl.ds(rbase, _PW)], comb_v, csem).wait()

    def add_chunk(buf):
        def row_body(k, carry):
            t = k // _PW
            r = k - t * _PW

            @plsc.parallel_loop(0, C, step=16, unroll=16)
            def _(off):
                plsc.addupdate(buf.at[t, r, pl.ds(off, 16)],
                               comb_v[r, pl.ds(off, 16)])
            return carry
        lax.fori_loop(0, _GRP * _PW, row_body, 0)

    ngrp = B // _GRP
    cp_in = [None] * ngrp
    cp_out = [None] * ngrp
    cp_in[0] = pltpu.async_copy(
        x_hbm.at[pl.ds(0, _GRP), pl.ds(rbase, _PW), :], bufs[0], isems[0])
    for g in range(ngrp):
        cp_in[g].wait()
        add_chunk(bufs[0])
        cp_out[g] = pltpu.async_copy(
            bufs[0], out_hbm.at[pl.ds(g * _GRP, _GRP), pl.ds(rbase, _PW), :],
            osems[0])
        if g + 1 < ngrp:
            cp_out[g].wait()
            cp_in[g + 1] = pltpu.async_copy(
                x_hbm.at[pl.ds((g + 1) * _GRP, _GRP), pl.ds(rbase, _PW), :],
                bufs[0], isems[0])
    cp_out[ngrp - 1].wait()


def _sc_add(x3, comb2):
    mesh = plsc.VectorSubcoreMesh(core_axis_name="c", subcore_axis_name="s")
    fn = pl.kernel(
        _sc_add_body,
        out_type=jax.ShapeDtypeStruct((B, H * W, C), jnp.float32),
        mesh=mesh,
        scratch_types=(
            [pltpu.VMEM((_PW, C), jnp.float32)]
            + [pltpu.VMEM((_GRP, _PW, C), jnp.float32)]
            + [pltpu.SemaphoreType.DMA] * 3
        ),
    )
    return fn(x3, comb2)


def kernel(x, y, base_prompt):
    y32 = y.astype(jnp.int32)
    base_jic = jnp.transpose(base_prompt, (1, 0, 2)).reshape(32, 32 * C).astype(jnp.bfloat16)
    expand = jnp.asarray(np.repeat(np.eye(32, dtype=np.float32), C, axis=1),
                         dtype=jnp.bfloat16)
    comb = _overlay(y32, base_jic, expand)
    out = _sc_add(x.reshape(B, H * W, C), comb)
    return out.reshape(B, H, W, C)


# speedup vs baseline: 1.0930x; 1.0930x over previous
import jax
import jax.numpy as jnp
import numpy as np
from jax import lax
from jax.experimental import pallas as pl
from jax.experimental.pallas import tpu as pltpu
from jax.experimental.pallas import tpu_sc as plsc

H = W = 32
C = 768
NBOX = 6
B = 8
_EPS32 = float(np.finfo(np.float32).eps)

_NW = 32
_PW = (H * W) // _NW
_CHUNK = _PW * C
_XWORDS = B * H * W * C
_GRP = 2


def _wmat(lo, hi):
    n = (hi - lo + 1).astype(jnp.float32)
    inv = 32.0 / n
    r = jax.lax.broadcasted_iota(jnp.int32, (32, 32), 1)
    i_in = jax.lax.broadcasted_iota(jnp.int32, (32, 32), 0).astype(jnp.float32)
    j = jnp.clip(r - lo, 0, 31).astype(jnp.float32)
    sample_f = (j + 0.5) * inv - 0.5
    wt = jnp.maximum(0.0, 1.0 - jnp.abs(sample_f - i_in) / inv)
    tot = jnp.sum(wt, axis=0, keepdims=True)
    wt = jnp.where(jnp.abs(tot) > 1000.0 * _EPS32,
                   wt / jnp.where(tot != 0.0, tot, 1.0), 0.0)
    wt = jnp.where((sample_f >= -0.5) & (sample_f <= 31.5), wt, 0.0)
    return wt


def _overlay_body(y_ref, base_ref, e_ref, comb_ref):
    rr = jax.lax.broadcasted_iota(jnp.int32, (H, W), 0)
    cc = jax.lax.broadcasted_iota(jnp.int32, (H, W), 1)
    applied = jnp.zeros((H, W), jnp.float32)
    comb = jnp.zeros((H, W, C), jnp.float32)
    gxs, gys, masks = [], [], []
    for i in range(NBOX):
        b0 = y_ref[i, 0]
        b1 = y_ref[i, 1]
        b2 = y_ref[i, 2]
        b3 = y_ref[i, 3]
        valid = ((b0 >= 0) & (b1 >= 0) & (b2 >= 0) & (b3 >= 0)).astype(jnp.float32)
        x1g = jnp.clip(jnp.floor(b0.astype(jnp.float32) * (1.0 / 16.0)), 0.0, 31.0)
        y1g = jnp.clip(jnp.floor(b1.astype(jnp.float32) * (1.0 / 16.0)), 0.0, 31.0)
        x2g = jnp.clip(jnp.floor(b2.astype(jnp.float32) * (1.0 / 16.0)), 0.0, 31.0)
        y2g = jnp.clip(jnp.floor(b3.astype(jnp.float32) * (1.0 / 16.0)), 0.0, 31.0)
        x_min = jnp.minimum(x1g, x2g).astype(jnp.int32)
        x_max = jnp.maximum(x1g, x2g).astype(jnp.int32)
        y_min = jnp.minimum(y1g, y2g).astype(jnp.int32)
        y_max = jnp.maximum(y1g, y2g).astype(jnp.int32)

        box_mask = ((rr >= y_min) & (rr <= y_max) &
                    (cc >= x_min) & (cc <= x_max)).astype(jnp.float32) * valid
        new_mask = box_mask * (1.0 - applied)
        applied = applied + new_mask
        masks.append(new_mask)
        gxs.append(_wmat(x_min, x_max))
        gys.append(_wmat(y_min, y_max))

    gxcat = jnp.concatenate(gxs, axis=1).astype(jnp.bfloat16)
    v = jax.lax.dot_general(gxcat, base_ref[...], (((0,), (0,)), ((), ())),
                            preferred_element_type=jnp.float32)
    v = v.astype(jnp.bfloat16)
    v4t = jnp.swapaxes(v.reshape(NBOX, W, 32, C), 1, 2)
    for i in range(NBOX):
        w = jax.lax.dot_general(gys[i].astype(jnp.bfloat16),
                                v4t[i].reshape(32, W * C),
                                (((0,), (0,)), ((), ())),
                                preferred_element_type=jnp.float32)
        m2 = jax.lax.dot_general(masks[i].astype(jnp.bfloat16), e_ref[...],
                                 (((1,), (0,)), ((), ())),
                                 preferred_element_type=jnp.float32)
        comb = comb + (w * m2).reshape(H, W, C)
    comb_ref[...] = comb.reshape(H * W, C)


def _overlay(y32, base_jic, expand):
    return pl.pallas_call(
        _overlay_body,
        grid=(1,),
        in_specs=[
            pl.BlockSpec(memory_space=pltpu.SMEM),
            pl.BlockSpec((32, 32 * C), lambda b: (0, 0)),
            pl.BlockSpec((32, 32 * C), lambda b: (0, 0)),
        ],
        out_specs=pl.BlockSpec((H * W, C), lambda b: (0, 0)),
        out_shape=jax.ShapeDtypeStruct((H * W, C), jnp.float32),
    )(y32, base_jic, expand)


def _sc_add_body(x_hbm, comb_hbm, out_hbm, comb_v, b0, b1,
                 csem, is0, is1, os0, os1):
    bufs = (b0, b1)
    isems = (is0, is1)
    osems = (os0, os1)
    cid = lax.axis_index("c")
    sid = lax.axis_index("s")
    wid = sid * 2 + cid
    rbase = wid * _PW

    pltpu.async_copy(comb_hbm.at[pl.ds(rbase, _PW)], comb_v, csem).wait()

    def add_chunk(buf):
        def row_body(k, carry):
            t = k // _PW
            r = k - t * _PW

            @plsc.parallel_loop(0, C, step=16, unroll=16)
            def _(off):
                plsc.addupdate(buf.at[t, r, pl.ds(off, 16)],
                               comb_v[r, pl.ds(off, 16)])
            return carry
        lax.fori_loop(0, _GRP * _PW, row_body, 0)

    ngrp = B // _GRP
    cp_in = [None] * ngrp
    cp_out = [None] * ngrp
    cp_in[0] = pltpu.async_copy(
        x_hbm.at[pl.ds(0, _GRP), pl.ds(rbase, _PW), :], bufs[0], isems[0])
    for g in range(ngrp):
        ng = g + 1
        if ng < ngrp:
            if ng >= 2:
                cp_out[ng - 2].wait()
            cp_in[ng] = pltpu.async_copy(
                x_hbm.at[pl.ds(ng * _GRP, _GRP), pl.ds(rbase, _PW), :],
                bufs[ng % 2], isems[ng % 2])
        cp_in[g].wait()
        add_chunk(bufs[g % 2])
        cp_out[g] = pltpu.async_copy(
            bufs[g % 2], out_hbm.at[pl.ds(g * _GRP, _GRP), pl.ds(rbase, _PW), :],
            osems[g % 2])
    for g in range(ngrp - 2, ngrp):
        cp_out[g].wait()


def _sc_add(x3, comb2):
    mesh = plsc.VectorSubcoreMesh(core_axis_name="c", subcore_axis_name="s")
    fn = pl.kernel(
        _sc_add_body,
        out_type=jax.ShapeDtypeStruct((B, H * W, C), jnp.float32),
        mesh=mesh,
        scratch_types=(
            [pltpu.VMEM((_PW, C), jnp.float32)]
            + [pltpu.VMEM((_GRP, _PW, C), jnp.float32)] * 2
            + [pltpu.SemaphoreType.DMA] * 5
        ),
    )
    return fn(x3, comb2)


def kernel(x, y, base_prompt):
    y32 = y.astype(jnp.int32)
    base_jic = jnp.transpose(base_prompt, (1, 0, 2)).reshape(32, 32 * C).astype(jnp.bfloat16)
    expand = jnp.asarray(np.repeat(np.eye(32, dtype=np.float32), C, axis=1),
                         dtype=jnp.bfloat16)
    comb = _overlay(y32, base_jic, expand)
    out = _sc_add(x.reshape(B, H * W, C), comb)
    return out.reshape(B, H, W, C)
